# Initial kernel scaffold; baseline (speedup 1.0000x reference)
import functools

import jax
import jax.numpy as jnp
from jax import lax
from jax.experimental import pallas as pl
from jax.experimental.pallas import tpu as pltpu
from jax.experimental.pallas import tpu_sc as plsc

_SCALE = 10.0
_D = 128
_B = 4096 * 50
_NC = 2
_NS = 16
_NW = _NC * _NS
_BPW = _B // _NW
_CHUNK = 128
_NCHUNK = _BPW // _CHUNK
_LANES = 16


def _scale_buf(buf):
    def row_body(r, carry):
        for k in range(_D // _LANES):
            sl = pl.ds(k * _LANES, _LANES)
            buf[r, sl] = buf[r, sl] * _SCALE
        return carry

    lax.fori_loop(0, _CHUNK, row_body, 0)


_mesh = plsc.VectorSubcoreMesh(core_axis_name="c", subcore_axis_name="s")


@functools.partial(
    pl.kernel,
    out_type=jax.ShapeDtypeStruct((_B, _D), jnp.float32),
    mesh=_mesh,
    scratch_types=[
        pltpu.VMEM((_NCHUNK, _CHUNK), jnp.int32),
        pltpu.VMEM((_CHUNK, _D), jnp.float32),
        pltpu.VMEM((_CHUNK, _D), jnp.float32),
        pltpu.SemaphoreType.DMA,
        pltpu.SemaphoreType.DMA,
    ],
)
def _gather_scale(table_hbm, idx_hbm, out_hbm, idx_v, buf0, buf1, sem0, sem1):
    wid = lax.axis_index("s") * _NC + lax.axis_index("c")
    pltpu.sync_copy(idx_hbm.at[pl.ds(wid * _NCHUNK, _NCHUNK)], idx_v)

    out_base = wid * _BPW

    pltpu.make_async_copy(table_hbm.at[idx_v.at[0]], buf0, sem0).start()
    pltpu.make_async_copy(table_hbm.at[idx_v.at[1]], buf1, sem1).start()

    def handle(c, buf, sem):
        pltpu.make_async_copy(table_hbm.at[idx_v.at[c]], buf, sem).wait()
        _scale_buf(buf)
        pltpu.sync_copy(buf, out_hbm.at[pl.ds(out_base + c * _CHUNK, _CHUNK)])

        @pl.when(c + 2 < _NCHUNK)
        def _():
            pltpu.make_async_copy(table_hbm.at[idx_v.at[c + 2]], buf, sem).start()

    def body(i, carry):
        handle(2 * i, buf0, sem0)
        handle(2 * i + 1, buf1, sem1)
        return carry

    lax.fori_loop(0, _NCHUNK // 2, body, 0)


def kernel(x, weight):
    idx = x.reshape(-1).astype(jnp.int32).reshape(_B // _CHUNK, _CHUNK)
    out = _gather_scale(weight, idx)
    return out.reshape(x.shape[0], x.shape[1], _D)


# speedup vs baseline: 2.8954x; 2.8954x over previous
import functools

import jax
import jax.numpy as jnp
from jax import lax
from jax.experimental import pallas as pl
from jax.experimental.pallas import tpu as pltpu
from jax.experimental.pallas import tpu_sc as plsc

_SCALE = 10.0
_D = 128
_B = 4096 * 50
_NC = 2
_NS = 16
_NW = _NC * _NS
_BPW = _B // _NW
_CHUNK = 128
_NCHUNK = _BPW // _CHUNK
_LANES = 16


def _scale_buf(buf):
    def row_body(r, carry):
        for k in range(_D // _LANES):
            sl = pl.ds(k * _LANES, _LANES)
            buf[r, sl] = buf[r, sl] * _SCALE
        return carry

    lax.fori_loop(0, _CHUNK, row_body, 0)


_mesh = plsc.VectorSubcoreMesh(core_axis_name="c", subcore_axis_name="s")


@functools.partial(
    pl.kernel,
    out_type=jax.ShapeDtypeStruct((_B, _D), jnp.float32),
    mesh=_mesh,
    scratch_types=[
        pltpu.VMEM((_NCHUNK, _CHUNK), jnp.int32),
        pltpu.VMEM((_CHUNK, _D), jnp.float32),
        pltpu.VMEM((_CHUNK, _D), jnp.float32),
        pltpu.SemaphoreType.DMA,
        pltpu.SemaphoreType.DMA,
    ],
)
def _gather_scale(table_hbm, idx_hbm, out_hbm, idx_v, buf0, buf1, sem0, sem1):
    wid = lax.axis_index("s") * _NC + lax.axis_index("c")
    pltpu.sync_copy(idx_hbm.at[wid], idx_v)

    out_base = wid * _BPW

    pltpu.make_async_copy(table_hbm.at[idx_v.at[0]], buf0, sem0).start()
    pltpu.make_async_copy(table_hbm.at[idx_v.at[1]], buf1, sem1).start()

    def handle(c, buf, sem):
        pltpu.make_async_copy(table_hbm.at[idx_v.at[c]], buf, sem).wait()
        _scale_buf(buf)
        pltpu.sync_copy(buf, out_hbm.at[pl.ds(out_base + c * _CHUNK, _CHUNK)])

        @pl.when(c + 2 < _NCHUNK)
        def _():
            pltpu.make_async_copy(table_hbm.at[idx_v.at[c + 2]], buf, sem).start()

    def body(i, carry):
        handle(2 * i, buf0, sem0)
        handle(2 * i + 1, buf1, sem1)
        return carry

    lax.fori_loop(0, _NCHUNK // 2, body, 0)


def kernel(x, weight):
    idx = x.reshape(-1).astype(jnp.int32).reshape(_NW, _NCHUNK, _CHUNK)
    out = _gather_scale(weight, idx)
    return out.reshape(x.shape[0], x.shape[1], _D)
